# baseline (speedup 1.0000x reference)
import functools

import jax
import jax.numpy as jnp
from jax import lax
from jax.experimental import pallas as pl
from jax.experimental.pallas import tpu as pltpu
from jax.experimental.pallas import tpu_sc as plsc

N = 10000
E = 160000
D = 300
H = 150
HP = 160
G = 256
NL = 5
BN_ = 1000
NBLK = N // BN_

NSUB = 16
EPT = E // NSUB
CH = 80
NCH = EPT // CH
STRIPE = N // NSUB

EPT2 = E // 32
CH2 = 100
NCH2 = EPT2 // CH2
SALT = 512


def _sc_scatter_sum(table, rc3, init, n_chunks, ch, width):
    mesh = plsc.VectorSubcoreMesh(core_axis_name="c", subcore_axis_name="s")

    @functools.partial(
        pl.kernel,
        mesh=mesh,
        out_type=jax.ShapeDtypeStruct((2 * N, width), jnp.float32),
        scratch_types=[
            pltpu.VMEM((3, 2, ch), jnp.int32),
            pltpu.VMEM((2, ch, width), jnp.float32),
            pltpu.VMEM_SHARED((N, width), jnp.float32),
            pltpu.SemaphoreType.DMA((3,)),
            pltpu.SemaphoreType.DMA((2,)),
            pltpu.SemaphoreType.DMA((2,)),
        ],
        compiler_params=pltpu.CompilerParams(use_tc_tiling_on_sc=False),
    )
    def body(t_hbm, rc3_hbm, init_hbm, out_hbm, idxb, stage, acc, sem_i,
             sem_g, sem_s):
        c = lax.axis_index("c")
        s = lax.axis_index("s")
        wid = c * NSUB + s
        pltpu.sync_copy(rc3_hbm.at[wid, 0], idxb.at[0])
        pltpu.sync_copy(init_hbm.at[pl.ds(c * N + s * STRIPE, STRIPE)],
                        acc.at[pl.ds(s * STRIPE, STRIPE)])
        plsc.subcore_barrier()
        pltpu.async_copy(t_hbm.at[idxb.at[0, 0]], stage.at[0], sem_g.at[0])
        pltpu.async_copy(rc3_hbm.at[wid, 1], idxb.at[1], sem_i.at[1])

        def step(j, carry):
            i3 = j % 3
            n3 = (j + 1) % 3
            cur = j % 2
            nxt = (j + 1) % 2

            @pl.when(j + 1 < n_chunks)
            def _():
                pltpu.make_async_copy(rc3_hbm.at[wid, j + 1], idxb.at[n3],
                                      sem_i.at[n3]).wait()

                @pl.when(j >= 1)
                def _():
                    pltpu.make_async_copy(
                        stage.at[nxt], acc.at[idxb.at[(j - 1) % 3, 1]],
                        sem_s.at[nxt]).wait()

                pltpu.async_copy(t_hbm.at[idxb.at[n3, 0]], stage.at[nxt],
                                 sem_g.at[nxt])

            pltpu.make_async_copy(t_hbm.at[idxb.at[i3, 0]], stage.at[cur],
                                  sem_g.at[cur]).wait()
            pltpu.async_copy(stage.at[cur], acc.at[idxb.at[i3, 1]],
                             sem_s.at[cur], add=True)

            @pl.when(j + 2 < n_chunks)
            def _():
                pltpu.async_copy(rc3_hbm.at[wid, j + 2], idxb.at[(j + 2) % 3],
                                 sem_i.at[(j + 2) % 3])

            return carry

        lax.fori_loop(0, n_chunks, step, 0)
        pltpu.make_async_copy(
            stage.at[(n_chunks - 2) % 2],
            acc.at[idxb.at[(n_chunks - 2) % 3, 1]],
            sem_s.at[(n_chunks - 2) % 2]).wait()
        pltpu.make_async_copy(
            stage.at[(n_chunks - 1) % 2],
            acc.at[idxb.at[(n_chunks - 1) % 3, 1]],
            sem_s.at[(n_chunks - 1) % 2]).wait()
        plsc.subcore_barrier()
        pltpu.sync_copy(acc.at[pl.ds(s * STRIPE, STRIPE)],
                        out_hbm.at[pl.ds(c * N + s * STRIPE, STRIPE)])

    return body(table, rc3, init)


def _tc_init(x0f, x1f, e1h, e2h):
    def body(x0_ref, x1_ref, e1_ref, e2_ref, out_ref):
        x0 = x0_ref[...]
        x1 = x1_ref[...]
        e1 = e1_ref[0]
        e2 = e2_ref[0]
        h = jnp.zeros((BN_, HP), jnp.float32)
        for j in range(3):
            h = h + jnp.where(x0 == float(j), 1.0, 0.0) * e1[j][None, :]
            h = h + jnp.where(x1 == float(j), 1.0, 0.0) * e2[j][None, :]
        out_ref[...] = h

    return pl.pallas_call(
        body,
        grid=(2, NBLK),
        in_specs=[
            pl.BlockSpec((BN_, 1), lambda c, i: (i, 0)),
            pl.BlockSpec((BN_, 1), lambda c, i: (i, 0)),
            pl.BlockSpec((1, 8, HP), lambda c, i: (c, 0, 0)),
            pl.BlockSpec((1, 8, HP), lambda c, i: (c, 0, 0)),
        ],
        out_specs=pl.BlockSpec((BN_, HP), lambda c, i: (c * NBLK + i, 0)),
        out_shape=jax.ShapeDtypeStruct((2 * N, HP), jnp.float32),
    )(x0f, x1f, e1h, e2h)


_PREC = lax.Precision.HIGHEST


def _tc_layer_fused(acc2, w4, s2, b_pp, g_pp, bt_pp, l, relu):
    def body(a0, a1, w, s0, s1, b, g, bt, out_ref, pre_scr, st_scr):
        p = pl.program_id(1)
        i = pl.program_id(2)

        @pl.when(p == 0)
        def _():
            mm = (jnp.dot(a0[...], w[0, 0], preferred_element_type=jnp.float32,
                          precision=_PREC)
                  + jnp.dot(a1[...], w[0, 1],
                            preferred_element_type=jnp.float32,
                            precision=_PREC))
            sv = s0[...] + s1[...]
            pre = mm + sv[:, l:l + 1] + b[0]
            pre_scr[pl.ds(i * BN_, BN_), :] = pre
            st = jnp.concatenate(
                [jnp.sum(pre, axis=0)[None, :],
                 jnp.sum(pre * pre, axis=0)[None, :],
                 jnp.zeros((6, HP), jnp.float32)], axis=0)

            @pl.when(i == 0)
            def _():
                st_scr[...] = st

            @pl.when(i > 0)
            def _():
                st_scr[...] += st

        @pl.when(p == 1)
        def _():
            xx = pre_scr[pl.ds(i * BN_, BN_), :]
            st = st_scr[...]
            mean = st[0] * (1.0 / N)
            var = st[1] * (1.0 / N) - mean * mean
            inv = lax.rsqrt(var + 1e-5)
            y = (xx - mean[None, :]) * inv[None, :] * g[0] + bt[0]
            if relu:
                y = jnp.maximum(y, 0.0)
            out_ref[...] = y

    return pl.pallas_call(
        body,
        grid=(2, 2, NBLK),
        in_specs=[
            pl.BlockSpec((BN_, HP), lambda c, p, i: (i * (1 - p), 0)),
            pl.BlockSpec((BN_, HP), lambda c, p, i: (NBLK + i * (1 - p), 0)),
            pl.BlockSpec((1, 2, HP, HP), lambda c, p, i: (c, 0, 0, 0)),
            pl.BlockSpec((BN_, 16), lambda c, p, i: (i * (1 - p), 0)),
            pl.BlockSpec((BN_, 16), lambda c, p, i: (NBLK + i * (1 - p), 0)),
            pl.BlockSpec((1, 1, HP), lambda c, p, i: (c, 0, 0)),
            pl.BlockSpec((1, 1, HP), lambda c, p, i: (c, 0, 0)),
            pl.BlockSpec((1, 1, HP), lambda c, p, i: (c, 0, 0)),
        ],
        out_specs=pl.BlockSpec((BN_, HP), lambda c, p, i: (c * NBLK + i * p, 0)),
        out_shape=jax.ShapeDtypeStruct((2 * N, HP), jnp.float32),
        scratch_shapes=[
            pltpu.VMEM((N, HP), jnp.float32),
            pltpu.VMEM((8, HP), jnp.float32),
        ],
    )(acc2, acc2, w4, s2, s2, b_pp, g_pp, bt_pp)


def _tc_layer_final(acc2, w4, s2, b_pp, g_pp, bt_pp, bf, fw_pp, fb,
                    w1, b1, w2, b2, l):
    def body(a0, a1, w, s0, s1, b, g, bt, bb, fw, fbr, w1r, b1r, w2r, b2r,
             hf_ref, out_ref, pre_scr, st_scr, su0, su1, cnt):
        cc = pl.program_id(0)
        p = pl.program_id(1)
        i = pl.program_id(2)

        @pl.when((cc == 0) & (p == 0) & (i == 0))
        def _():
            su0[...] = jnp.zeros_like(su0)
            su1[...] = jnp.zeros_like(su1)
            cnt[...] = jnp.zeros_like(cnt)

        @pl.when(p == 0)
        def _():
            mm = (jnp.dot(a0[...], w[0, 0], preferred_element_type=jnp.float32,
                          precision=_PREC)
                  + jnp.dot(a1[...], w[0, 1],
                            preferred_element_type=jnp.float32,
                            precision=_PREC))
            sv = s0[...] + s1[...]
            pre = mm + sv[:, l:l + 1] + b[0]
            pre_scr[pl.ds(i * BN_, BN_), :] = pre
            st = jnp.concatenate(
                [jnp.sum(pre, axis=0)[None, :],
                 jnp.sum(pre * pre, axis=0)[None, :],
                 jnp.zeros((6, HP), jnp.float32)], axis=0)

            @pl.when(i == 0)
            def _():
                st_scr[...] = st

            @pl.when(i > 0)
            def _():
                st_scr[...] += st

        @pl.when(p == 1)
        def _():
            xx = pre_scr[pl.ds(i * BN_, BN_), :]
            st = st_scr[...]
            mean = st[0] * (1.0 / N)
            var = st[1] * (1.0 / N) - mean * mean
            inv = lax.rsqrt(var + 1e-5)
            y = (xx - mean[None, :]) * inv[None, :] * g[0] + bt[0]
            b_ = bb[...]
            gid = lax.broadcasted_iota(jnp.int32, (1, G), 1).astype(jnp.float32)
            oh = jnp.where(b_ == gid, 1.0, 0.0)
            dn = (((0,), (0,)), ((), ()))

            @pl.when(cc == 0)
            def _():
                su0[...] += lax.dot_general(
                    oh, y, dn, preferred_element_type=jnp.float32,
                    precision=_PREC)
                cnt[...] += lax.dot_general(
                    oh, jnp.ones((BN_, 1), jnp.float32), dn,
                    preferred_element_type=jnp.float32, precision=_PREC)

            @pl.when(cc == 1)
            def _():
                su1[...] += lax.dot_general(
                    oh, y, dn, preferred_element_type=jnp.float32,
                    precision=_PREC)

        @pl.when((cc == 1) & (p == 1) & (i == NBLK - 1))
        def _():
            m = jnp.maximum(cnt[...], 1.0)
            hg0 = su0[...] / m
            hg1 = su1[...] / m
            hf = (jnp.dot(hg0, fw[0], preferred_element_type=jnp.float32,
                          precision=_PREC)
                  + jnp.dot(hg1, fw[1], preferred_element_type=jnp.float32,
                            precision=_PREC)
                  + fbr[...])
            z = jnp.maximum(
                jnp.dot(hf, w1r[...], preferred_element_type=jnp.float32,
                        precision=_PREC) + b1r[...], 0.0)
            o = (jnp.dot(z, w2r[...], preferred_element_type=jnp.float32,
                         precision=_PREC) + b2r[...])
            hf_ref[...] = hf
            out_ref[...] = o

    return pl.pallas_call(
        body,
        grid=(2, 2, NBLK),
        in_specs=[
            pl.BlockSpec((BN_, HP), lambda c, p, i: (i * (1 - p), 0)),
            pl.BlockSpec((BN_, HP), lambda c, p, i: (NBLK + i * (1 - p), 0)),
            pl.BlockSpec((1, 2, HP, HP), lambda c, p, i: (c, 0, 0, 0)),
            pl.BlockSpec((BN_, 16), lambda c, p, i: (i * (1 - p), 0)),
            pl.BlockSpec((BN_, 16), lambda c, p, i: (NBLK + i * (1 - p), 0)),
            pl.BlockSpec((1, 1, HP), lambda c, p, i: (c, 0, 0)),
            pl.BlockSpec((1, 1, HP), lambda c, p, i: (c, 0, 0)),
            pl.BlockSpec((1, 1, HP), lambda c, p, i: (c, 0, 0)),
            pl.BlockSpec((BN_, 1), lambda c, p, i: (i * p, 0)),
            pl.BlockSpec((2, HP, G), lambda c, p, i: (0, 0, 0)),
            pl.BlockSpec((1, G), lambda c, p, i: (0, 0)),
            pl.BlockSpec((G, G), lambda c, p, i: (0, 0)),
            pl.BlockSpec((1, G), lambda c, p, i: (0, 0)),
            pl.BlockSpec((G, G // 2), lambda c, p, i: (0, 0)),
            pl.BlockSpec((1, G // 2), lambda c, p, i: (0, 0)),
        ],
        out_specs=[
            pl.BlockSpec((G, G), lambda c, p, i: (0, 0)),
            pl.BlockSpec((G, G // 2), lambda c, p, i: (0, 0)),
        ],
        out_shape=[
            jax.ShapeDtypeStruct((G, G), jnp.float32),
            jax.ShapeDtypeStruct((G, G // 2), jnp.float32),
        ],
        scratch_shapes=[
            pltpu.VMEM((N, HP), jnp.float32),
            pltpu.VMEM((8, HP), jnp.float32),
            pltpu.VMEM((G, HP), jnp.float32),
            pltpu.VMEM((G, HP), jnp.float32),
            pltpu.VMEM((G, 1), jnp.float32),
        ],
    )(acc2, acc2, w4, s2, s2, b_pp, g_pp, bt_pp, bf, fw_pp, fb,
      w1, b1, w2, b2)


def _halves(v):
    out = jnp.zeros((2, 1, HP), jnp.float32)
    out = out.at[0, 0, :H].set(v[:H]).at[1, 0, :H].set(v[H:])
    return out


def kernel(x, edge_index, edge_attr, batch, params):
    f32, i32 = jnp.float32, jnp.int32
    x = x.astype(i32)
    row = edge_index[0].astype(i32)
    col = edge_index[1].astype(i32)
    ea0 = edge_attr[:, 0].astype(i32)
    ea1 = edge_attr[:, 1].astype(i32)
    batchf = batch.astype(f32)[:, None]
    x0f = x[:, 0].astype(f32)[:, None]
    x1f = x[:, 1].astype(f32)[:, None]
    row_r = jnp.stack([row, row + N]).reshape(2, NSUB, NCH, CH)
    col_r = jnp.broadcast_to(col.reshape(1, NSUB, NCH, CH), (2, NSUB, NCH, CH))
    rc3 = jnp.stack([row_r, col_r], axis=3).reshape(2 * NSUB, NCH, 2, CH)
    kidx = ea0 * 3 + ea1 + 16 * (jnp.arange(E, dtype=i32) % SALT)
    k_r = kidx.reshape(2 * NSUB, NCH2, CH2)
    colk_r = col.reshape(2 * NSUB, NCH2, CH2)
    rc3s = jnp.stack([k_r, colk_r], axis=2)

    p = params
    la = p['layers']

    def halves_pad(m, rows_pad):
        out = jnp.zeros((2, rows_pad, HP), f32)
        out = out.at[0, :m.shape[0], :H].set(m[:, :H])
        out = out.at[1, :m.shape[0], :H].set(m[:, H:D])
        return out

    e1h = halves_pad(p['emb1'][:3], 8)
    e2h = halves_pad(p['emb2'][:3], 8)

    ee1 = jnp.stack([lp['ee1'][:, 0] for lp in la])
    ee2 = jnp.stack([lp['ee2'][:, 0] for lp in la])
    kk = jnp.arange(9)
    t9 = (ee1[:, kk // 3] + ee2[:, kk % 3]).T
    t16 = jnp.zeros((16, 16), f32).at[:9, :5].set(t9)
    t16r = jnp.broadcast_to(t16[None], (SALT, 16, 16)).reshape(SALT * 16, 16)

    zeros_s = jnp.zeros((2 * N, 16), f32)

    h2 = _tc_init(x0f, x1f, e1h, e2h)
    s2 = _sc_scatter_sum(t16r, rc3s, zeros_s, NCH2, CH2, 16)

    fW = p['feat_W']
    fw_pp = jnp.zeros((2, HP, G), f32).at[0, :H].set(fW[:H]).at[1, :H].set(fW[H:])

    for l in range(NL):
        lp = la[l]
        W = lp['W']
        w4 = jnp.zeros((2, 2, HP, HP), f32)
        w4 = (w4.at[0, 0, :H, :H].set(W[:H, :H])
                .at[0, 1, :H, :H].set(W[H:, :H])
                .at[1, 0, :H, :H].set(W[:H, H:])
                .at[1, 1, :H, :H].set(W[H:, H:]))
        selfc = lp['ee1'][4, 0] + lp['ee2'][0, 0]
        b_pp = _halves(lp['b'] + selfc)
        g_pp = _halves(lp['gamma'])
        bt_pp = _halves(lp['beta'])

        acc2 = _sc_scatter_sum(h2, rc3, h2, NCH, CH, HP)
        if l != NL - 1:
            h2 = _tc_layer_fused(acc2, w4, s2, b_pp, g_pp, bt_pp, l,
                                 relu=True)
        else:
            hf, out = _tc_layer_final(
                acc2, w4, s2, b_pp, g_pp, bt_pp, batchf, fw_pp,
                p['feat_b'][None, :], p['out_W1'], p['out_b1'][None, :],
                p['out_W2'], p['out_b2'][None, :], l)
    return hf, out


# speedup vs baseline: 1.0118x; 1.0118x over previous
import functools

import jax
import jax.numpy as jnp
from jax import lax
from jax.experimental import pallas as pl
from jax.experimental.pallas import tpu as pltpu
from jax.experimental.pallas import tpu_sc as plsc

N = 10000
E = 160000
D = 300
H = 150
HP = 160
G = 256
NL = 5
BN_ = 2000
NBLK = N // BN_

NSUB = 16
EPT = E // NSUB
CH = 80
NCH = EPT // CH
STRIPE = N // NSUB

EPT2 = E // 32
CH2 = 100
NCH2 = EPT2 // CH2
SALT = 512


def _sc_scatter_sum(table, rc3, init, n_chunks, ch, width):
    mesh = plsc.VectorSubcoreMesh(core_axis_name="c", subcore_axis_name="s")

    @functools.partial(
        pl.kernel,
        mesh=mesh,
        out_type=jax.ShapeDtypeStruct((2 * N, width), jnp.float32),
        scratch_types=[
            pltpu.VMEM((3, 2, ch), jnp.int32),
            pltpu.VMEM((2, ch, width), jnp.float32),
            pltpu.VMEM_SHARED((N, width), jnp.float32),
            pltpu.SemaphoreType.DMA((3,)),
            pltpu.SemaphoreType.DMA((2,)),
            pltpu.SemaphoreType.DMA((2,)),
        ],
        compiler_params=pltpu.CompilerParams(use_tc_tiling_on_sc=False),
    )
    def body(t_hbm, rc3_hbm, init_hbm, out_hbm, idxb, stage, acc, sem_i,
             sem_g, sem_s):
        c = lax.axis_index("c")
        s = lax.axis_index("s")
        wid = c * NSUB + s
        pltpu.sync_copy(rc3_hbm.at[wid, 0], idxb.at[0])
        pltpu.sync_copy(init_hbm.at[pl.ds(c * N + s * STRIPE, STRIPE)],
                        acc.at[pl.ds(s * STRIPE, STRIPE)])
        plsc.subcore_barrier()
        pltpu.async_copy(t_hbm.at[idxb.at[0, 0]], stage.at[0], sem_g.at[0])
        pltpu.async_copy(rc3_hbm.at[wid, 1], idxb.at[1], sem_i.at[1])

        def step(j, carry):
            i3 = j % 3
            n3 = (j + 1) % 3
            cur = j % 2
            nxt = (j + 1) % 2

            @pl.when(j + 1 < n_chunks)
            def _():
                pltpu.make_async_copy(rc3_hbm.at[wid, j + 1], idxb.at[n3],
                                      sem_i.at[n3]).wait()

                @pl.when(j >= 1)
                def _():
                    pltpu.make_async_copy(
                        stage.at[nxt], acc.at[idxb.at[(j - 1) % 3, 1]],
                        sem_s.at[nxt]).wait()

                pltpu.async_copy(t_hbm.at[idxb.at[n3, 0]], stage.at[nxt],
                                 sem_g.at[nxt])

            pltpu.make_async_copy(t_hbm.at[idxb.at[i3, 0]], stage.at[cur],
                                  sem_g.at[cur]).wait()
            pltpu.async_copy(stage.at[cur], acc.at[idxb.at[i3, 1]],
                             sem_s.at[cur], add=True)

            @pl.when(j + 2 < n_chunks)
            def _():
                pltpu.async_copy(rc3_hbm.at[wid, j + 2], idxb.at[(j + 2) % 3],
                                 sem_i.at[(j + 2) % 3])

            return carry

        lax.fori_loop(0, n_chunks, step, 0)
        pltpu.make_async_copy(
            stage.at[(n_chunks - 2) % 2],
            acc.at[idxb.at[(n_chunks - 2) % 3, 1]],
            sem_s.at[(n_chunks - 2) % 2]).wait()
        pltpu.make_async_copy(
            stage.at[(n_chunks - 1) % 2],
            acc.at[idxb.at[(n_chunks - 1) % 3, 1]],
            sem_s.at[(n_chunks - 1) % 2]).wait()
        plsc.subcore_barrier()
        pltpu.sync_copy(acc.at[pl.ds(s * STRIPE, STRIPE)],
                        out_hbm.at[pl.ds(c * N + s * STRIPE, STRIPE)])

    return body(table, rc3, init)


def _tc_init(x0f, x1f, e1h, e2h):
    def body(x0_ref, x1_ref, e1_ref, e2_ref, out_ref):
        x0 = x0_ref[...]
        x1 = x1_ref[...]
        e1 = e1_ref[0]
        e2 = e2_ref[0]
        h = jnp.zeros((BN_, HP), jnp.float32)
        for j in range(3):
            h = h + jnp.where(x0 == float(j), 1.0, 0.0) * e1[j][None, :]
            h = h + jnp.where(x1 == float(j), 1.0, 0.0) * e2[j][None, :]
        out_ref[...] = h

    return pl.pallas_call(
        body,
        grid=(2, NBLK),
        in_specs=[
            pl.BlockSpec((BN_, 1), lambda c, i: (i, 0)),
            pl.BlockSpec((BN_, 1), lambda c, i: (i, 0)),
            pl.BlockSpec((1, 8, HP), lambda c, i: (c, 0, 0)),
            pl.BlockSpec((1, 8, HP), lambda c, i: (c, 0, 0)),
        ],
        out_specs=pl.BlockSpec((BN_, HP), lambda c, i: (c * NBLK + i, 0)),
        out_shape=jax.ShapeDtypeStruct((2 * N, HP), jnp.float32),
    )(x0f, x1f, e1h, e2h)


_PREC = lax.Precision.HIGHEST


def _tc_layer_fused(acc2, w4, s2, b_pp, g_pp, bt_pp, l, relu):
    def body(a0, a1, w, s0, s1, b, g, bt, out_ref, pre_scr, st_scr):
        p = pl.program_id(1)
        i = pl.program_id(2)

        @pl.when(p == 0)
        def _():
            mm = (jnp.dot(a0[...], w[0, 0], preferred_element_type=jnp.float32,
                          precision=_PREC)
                  + jnp.dot(a1[...], w[0, 1],
                            preferred_element_type=jnp.float32,
                            precision=_PREC))
            sv = s0[...] + s1[...]
            pre = mm + sv[:, l:l + 1] + b[0]
            pre_scr[pl.ds(i * BN_, BN_), :] = pre
            st = jnp.concatenate(
                [jnp.sum(pre, axis=0)[None, :],
                 jnp.sum(pre * pre, axis=0)[None, :],
                 jnp.zeros((6, HP), jnp.float32)], axis=0)

            @pl.when(i == 0)
            def _():
                st_scr[...] = st

            @pl.when(i > 0)
            def _():
                st_scr[...] += st

        @pl.when(p == 1)
        def _():
            xx = pre_scr[pl.ds(i * BN_, BN_), :]
            st = st_scr[...]
            mean = st[0] * (1.0 / N)
            var = st[1] * (1.0 / N) - mean * mean
            inv = lax.rsqrt(var + 1e-5)
            y = (xx - mean[None, :]) * inv[None, :] * g[0] + bt[0]
            if relu:
                y = jnp.maximum(y, 0.0)
            out_ref[...] = y

    return pl.pallas_call(
        body,
        grid=(2, 2, NBLK),
        in_specs=[
            pl.BlockSpec((BN_, HP), lambda c, p, i: (i * (1 - p), 0)),
            pl.BlockSpec((BN_, HP), lambda c, p, i: (NBLK + i * (1 - p), 0)),
            pl.BlockSpec((1, 2, HP, HP), lambda c, p, i: (c, 0, 0, 0)),
            pl.BlockSpec((BN_, 16), lambda c, p, i: (i * (1 - p), 0)),
            pl.BlockSpec((BN_, 16), lambda c, p, i: (NBLK + i * (1 - p), 0)),
            pl.BlockSpec((1, 1, HP), lambda c, p, i: (c, 0, 0)),
            pl.BlockSpec((1, 1, HP), lambda c, p, i: (c, 0, 0)),
            pl.BlockSpec((1, 1, HP), lambda c, p, i: (c, 0, 0)),
        ],
        out_specs=pl.BlockSpec((BN_, HP), lambda c, p, i: (c * NBLK + i * p, 0)),
        out_shape=jax.ShapeDtypeStruct((2 * N, HP), jnp.float32),
        scratch_shapes=[
            pltpu.VMEM((N, HP), jnp.float32),
            pltpu.VMEM((8, HP), jnp.float32),
        ],
    )(acc2, acc2, w4, s2, s2, b_pp, g_pp, bt_pp)


def _tc_layer_final(acc2, w4, s2, b_pp, g_pp, bt_pp, bf, fw_pp, fb,
                    w1, b1, w2, b2, l):
    def body(a0, a1, w, s0, s1, b, g, bt, bb, fw, fbr, w1r, b1r, w2r, b2r,
             hf_ref, out_ref, pre_scr, st_scr, su0, su1, cnt):
        cc = pl.program_id(0)
        p = pl.program_id(1)
        i = pl.program_id(2)

        @pl.when((cc == 0) & (p == 0) & (i == 0))
        def _():
            su0[...] = jnp.zeros_like(su0)
            su1[...] = jnp.zeros_like(su1)
            cnt[...] = jnp.zeros_like(cnt)

        @pl.when(p == 0)
        def _():
            mm = (jnp.dot(a0[...], w[0, 0], preferred_element_type=jnp.float32,
                          precision=_PREC)
                  + jnp.dot(a1[...], w[0, 1],
                            preferred_element_type=jnp.float32,
                            precision=_PREC))
            sv = s0[...] + s1[...]
            pre = mm + sv[:, l:l + 1] + b[0]
            pre_scr[pl.ds(i * BN_, BN_), :] = pre
            st = jnp.concatenate(
                [jnp.sum(pre, axis=0)[None, :],
                 jnp.sum(pre * pre, axis=0)[None, :],
                 jnp.zeros((6, HP), jnp.float32)], axis=0)

            @pl.when(i == 0)
            def _():
                st_scr[...] = st

            @pl.when(i > 0)
            def _():
                st_scr[...] += st

        @pl.when(p == 1)
        def _():
            xx = pre_scr[pl.ds(i * BN_, BN_), :]
            st = st_scr[...]
            mean = st[0] * (1.0 / N)
            var = st[1] * (1.0 / N) - mean * mean
            inv = lax.rsqrt(var + 1e-5)
            y = (xx - mean[None, :]) * inv[None, :] * g[0] + bt[0]
            b_ = bb[...]
            gid = lax.broadcasted_iota(jnp.int32, (1, G), 1).astype(jnp.float32)
            oh = jnp.where(b_ == gid, 1.0, 0.0)
            dn = (((0,), (0,)), ((), ()))

            @pl.when(cc == 0)
            def _():
                su0[...] += lax.dot_general(
                    oh, y, dn, preferred_element_type=jnp.float32,
                    precision=_PREC)
                cnt[...] += lax.dot_general(
                    oh, jnp.ones((BN_, 1), jnp.float32), dn,
                    preferred_element_type=jnp.float32, precision=_PREC)

            @pl.when(cc == 1)
            def _():
                su1[...] += lax.dot_general(
                    oh, y, dn, preferred_element_type=jnp.float32,
                    precision=_PREC)

        @pl.when((cc == 1) & (p == 1) & (i == NBLK - 1))
        def _():
            m = jnp.maximum(cnt[...], 1.0)
            hg0 = su0[...] / m
            hg1 = su1[...] / m
            hf = (jnp.dot(hg0, fw[0], preferred_element_type=jnp.float32,
                          precision=_PREC)
                  + jnp.dot(hg1, fw[1], preferred_element_type=jnp.float32,
                            precision=_PREC)
                  + fbr[...])
            z = jnp.maximum(
                jnp.dot(hf, w1r[...], preferred_element_type=jnp.float32,
                        precision=_PREC) + b1r[...], 0.0)
            o = (jnp.dot(z, w2r[...], preferred_element_type=jnp.float32,
                         precision=_PREC) + b2r[...])
            hf_ref[...] = hf
            out_ref[...] = o

    return pl.pallas_call(
        body,
        grid=(2, 2, NBLK),
        in_specs=[
            pl.BlockSpec((BN_, HP), lambda c, p, i: (i * (1 - p), 0)),
            pl.BlockSpec((BN_, HP), lambda c, p, i: (NBLK + i * (1 - p), 0)),
            pl.BlockSpec((1, 2, HP, HP), lambda c, p, i: (c, 0, 0, 0)),
            pl.BlockSpec((BN_, 16), lambda c, p, i: (i * (1 - p), 0)),
            pl.BlockSpec((BN_, 16), lambda c, p, i: (NBLK + i * (1 - p), 0)),
            pl.BlockSpec((1, 1, HP), lambda c, p, i: (c, 0, 0)),
            pl.BlockSpec((1, 1, HP), lambda c, p, i: (c, 0, 0)),
            pl.BlockSpec((1, 1, HP), lambda c, p, i: (c, 0, 0)),
            pl.BlockSpec((BN_, 1), lambda c, p, i: (i * p, 0)),
            pl.BlockSpec((2, HP, G), lambda c, p, i: (0, 0, 0)),
            pl.BlockSpec((1, G), lambda c, p, i: (0, 0)),
            pl.BlockSpec((G, G), lambda c, p, i: (0, 0)),
            pl.BlockSpec((1, G), lambda c, p, i: (0, 0)),
            pl.BlockSpec((G, G // 2), lambda c, p, i: (0, 0)),
            pl.BlockSpec((1, G // 2), lambda c, p, i: (0, 0)),
        ],
        out_specs=[
            pl.BlockSpec((G, G), lambda c, p, i: (0, 0)),
            pl.BlockSpec((G, G // 2), lambda c, p, i: (0, 0)),
        ],
        out_shape=[
            jax.ShapeDtypeStruct((G, G), jnp.float32),
            jax.ShapeDtypeStruct((G, G // 2), jnp.float32),
        ],
        scratch_shapes=[
            pltpu.VMEM((N, HP), jnp.float32),
            pltpu.VMEM((8, HP), jnp.float32),
            pltpu.VMEM((G, HP), jnp.float32),
            pltpu.VMEM((G, HP), jnp.float32),
            pltpu.VMEM((G, 1), jnp.float32),
        ],
    )(acc2, acc2, w4, s2, s2, b_pp, g_pp, bt_pp, bf, fw_pp, fb,
      w1, b1, w2, b2)


def _halves(v):
    out = jnp.zeros((2, 1, HP), jnp.float32)
    out = out.at[0, 0, :H].set(v[:H]).at[1, 0, :H].set(v[H:])
    return out


def kernel(x, edge_index, edge_attr, batch, params):
    f32, i32 = jnp.float32, jnp.int32
    x = x.astype(i32)
    row = edge_index[0].astype(i32)
    col = edge_index[1].astype(i32)
    ea0 = edge_attr[:, 0].astype(i32)
    ea1 = edge_attr[:, 1].astype(i32)
    batchf = batch.astype(f32)[:, None]
    x0f = x[:, 0].astype(f32)[:, None]
    x1f = x[:, 1].astype(f32)[:, None]
    row_r = jnp.stack([row, row + N]).reshape(2, NSUB, NCH, CH)
    col_r = jnp.broadcast_to(col.reshape(1, NSUB, NCH, CH), (2, NSUB, NCH, CH))
    rc3 = jnp.stack([row_r, col_r], axis=3).reshape(2 * NSUB, NCH, 2, CH)
    kidx = ea0 * 3 + ea1 + 16 * (jnp.arange(E, dtype=i32) % SALT)
    k_r = kidx.reshape(2 * NSUB, NCH2, CH2)
    colk_r = col.reshape(2 * NSUB, NCH2, CH2)
    rc3s = jnp.stack([k_r, colk_r], axis=2)

    p = params
    la = p['layers']

    def halves_pad(m, rows_pad):
        out = jnp.zeros((2, rows_pad, HP), f32)
        out = out.at[0, :m.shape[0], :H].set(m[:, :H])
        out = out.at[1, :m.shape[0], :H].set(m[:, H:D])
        return out

    e1h = halves_pad(p['emb1'][:3], 8)
    e2h = halves_pad(p['emb2'][:3], 8)

    ee1 = jnp.stack([lp['ee1'][:, 0] for lp in la])
    ee2 = jnp.stack([lp['ee2'][:, 0] for lp in la])
    kk = jnp.arange(9)
    t9 = (ee1[:, kk // 3] + ee2[:, kk % 3]).T
    t16 = jnp.zeros((16, 16), f32).at[:9, :5].set(t9)
    t16r = jnp.broadcast_to(t16[None], (SALT, 16, 16)).reshape(SALT * 16, 16)

    zeros_s = jnp.zeros((2 * N, 16), f32)

    h2 = _tc_init(x0f, x1f, e1h, e2h)
    s2 = _sc_scatter_sum(t16r, rc3s, zeros_s, NCH2, CH2, 16)

    fW = p['feat_W']
    fw_pp = jnp.zeros((2, HP, G), f32).at[0, :H].set(fW[:H]).at[1, :H].set(fW[H:])

    for l in range(NL):
        lp = la[l]
        W = lp['W']
        w4 = jnp.zeros((2, 2, HP, HP), f32)
        w4 = (w4.at[0, 0, :H, :H].set(W[:H, :H])
                .at[0, 1, :H, :H].set(W[H:, :H])
                .at[1, 0, :H, :H].set(W[:H, H:])
                .at[1, 1, :H, :H].set(W[H:, H:]))
        selfc = lp['ee1'][4, 0] + lp['ee2'][0, 0]
        b_pp = _halves(lp['b'] + selfc)
        g_pp = _halves(lp['gamma'])
        bt_pp = _halves(lp['beta'])

        acc2 = _sc_scatter_sum(h2, rc3, h2, NCH, CH, HP)
        if l != NL - 1:
            h2 = _tc_layer_fused(acc2, w4, s2, b_pp, g_pp, bt_pp, l,
                                 relu=True)
        else:
            hf, out = _tc_layer_final(
                acc2, w4, s2, b_pp, g_pp, bt_pp, batchf, fw_pp,
                p['feat_b'][None, :], p['out_W1'], p['out_b1'][None, :],
                p['out_W2'], p['out_b2'][None, :], l)
    return hf, out
